# Initial kernel scaffold; baseline (speedup 1.0000x reference)
import jax
import jax.numpy as jnp
from jax.experimental import pallas as pl


def kernel(x, edge_index, edge_type, batch, W1, root1, b1, W2, root2, b2):
    raise NotImplementedError("write your pallas kernel here")


# speedup vs baseline: 10.3879x; 10.3879x over previous
import functools

import jax
import jax.numpy as jnp
from jax import lax
from jax.experimental import pallas as pl
from jax.experimental.pallas import tpu as pltpu
from jax.experimental.pallas import tpu_sc as plsc

_N = 50000
_E = 800000
_G = 64

_B = 128
_CHUNK = 2000
_NV = _CHUNK // 16
_EPT = _E // 16
_NCH = _EPT // _CHUNK
_RCH = 128


def _make_sc_agg(D, NC, PP):
    DUMP = NC * 4
    ROWS_ACC = NC * 4 + _B
    RT = (NC * 4) // 16
    NRC = RT // _RCH
    OUT_ROWS = 2 * PP * NC * 4
    NBMAX = (_CHUNK + 2 * _B) // _B + 1

    mesh = plsc.VectorSubcoreMesh(core_axis_name="c", subcore_axis_name="s")

    @functools.partial(
        pl.kernel,
        out_type=jax.ShapeDtypeStruct((OUT_ROWS, D), jnp.float32),
        mesh=mesh,
        compiler_params=pltpu.CompilerParams(
            needs_layout_passes=False, use_tc_tiling_on_sc=False),
        scratch_types=[
            pltpu.VMEM((_CHUNK,), jnp.int32),
            pltpu.VMEM((_CHUNK,), jnp.int32),
            pltpu.VMEM((_CHUNK,), jnp.int32),
            pltpu.VMEM((_CHUNK + 2 * _B,), jnp.int32),
            pltpu.VMEM((NBMAX, _B), jnp.int32),
            pltpu.VMEM((_B, D), jnp.float32),
            pltpu.VMEM((_RCH, D), jnp.float32),
            pltpu.VMEM_SHARED((ROWS_ACC, D), jnp.float32),
            pltpu.SemaphoreType.DMA,
        ],
    )
    def agg(src_hbm, dst_hbm, typ_hbm, table_hbm, out_hbm,
            srcb, dstb, typb, csrc, cidx, rows, zbuf, acc, sem):
        cid = lax.axis_index("c")
        sid = lax.axis_index("s")
        lanes = lax.iota(jnp.int32, 16)
        z16 = jnp.zeros((16,), jnp.float32)

        def _zb(i, carry):
            for j in range(D // 16):
                zbuf[i, pl.ds(j * 16, 16)] = z16
            return carry
        lax.fori_loop(0, _RCH, _zb, 0)

        def one_partition(p, carry):
            lo = (cid * PP + p) * NC

            def _z(k, c):
                pltpu.sync_copy(zbuf, acc.at[pl.ds(sid * RT + k * _RCH, _RCH)])
                return c
            lax.fori_loop(0, NRC, _z, 0)
            plsc.subcore_barrier()

            def _chunk(ch, c):
                eb = sid * _EPT + ch * _CHUNK
                pltpu.sync_copy(src_hbm.at[pl.ds(eb, _CHUNK)],
                                srcb.at[pl.ds(0, _CHUNK)])
                pltpu.sync_copy(dst_hbm.at[pl.ds(eb, _CHUNK)],
                                dstb.at[pl.ds(0, _CHUNK)])
                pltpu.sync_copy(typ_hbm.at[pl.ds(eb, _CHUNK)],
                                typb.at[pl.ds(0, _CHUNK)])

                def _cvec(i, n):
                    off = i * 16
                    d = dstb[pl.ds(off, 16)]
                    s = srcb[pl.ds(off, 16)]
                    t = typb[pl.ds(off, 16)]
                    dl = d - lo
                    m = (dl >= 0) & (dl < NC)
                    ai = dl * 4 + t
                    cs = plsc.cumsum(jnp.ones((16,), jnp.int32), mask=m)
                    pos = n + cs - 1
                    plsc.store_scatter(csrc, [pos], s, mask=m)
                    plsc.store_scatter(
                        cidx, [pos >> 7, pos & (_B - 1)], ai, mask=m)
                    pc = plsc.all_reduce_population_count(m)
                    return n + jnp.max(pc)
                n = lax.fori_loop(0, _NV, _cvec, jnp.int32(0))

                for k in range(_B // 16):
                    pidx = n + k * 16 + lanes
                    plsc.store_scatter(csrc, [pidx], k * 16 + lanes)
                    plsc.store_scatter(cidx,
                                       [pidx >> 7, pidx & (_B - 1)],
                                       DUMP + k * 16 + lanes)

                nb = (n + (_B - 1)) // _B

                def _gs(b, c2):
                    pltpu.async_copy(
                        table_hbm.at[csrc.at[pl.ds(b * _B, _B)]],
                        rows, sem).wait()
                    pltpu.sync_copy(rows, acc.at[cidx.at[b]], add=True)
                    return c2
                lax.fori_loop(0, nb, _gs, 0)
                return c
            lax.fori_loop(0, _NCH, _chunk, 0)
            plsc.subcore_barrier()

            def _ro(k, c):
                lr = sid * RT + k * _RCH
                gr = (cid * PP + p) * NC * 4 + lr
                pltpu.sync_copy(acc.at[pl.ds(lr, _RCH)],
                                rows.at[pl.ds(0, _RCH)])
                pltpu.sync_copy(rows.at[pl.ds(0, _RCH)],
                                out_hbm.at[pl.ds(gr, _RCH)])
                return c
            lax.fori_loop(0, NRC, _ro, 0)
            plsc.subcore_barrier()
            return carry
        lax.fori_loop(0, PP, one_partition, 0)

    return agg


_make_sc_agg = functools.lru_cache(maxsize=None)(_make_sc_agg)

_BLK = 1000
_NB = _N // _BLK


def _tc1_body(x_ref, a_ref, r_ref, w_ref, h_ref, inv_ref):
    a = a_ref[...]
    jr = lax.broadcasted_iota(jnp.int32, (64, 4), 0)
    rc = lax.broadcasted_iota(jnp.int32, (64, 4), 1)
    sel = (jr == rc * 16 + 15).astype(jnp.float32)
    cnt = jnp.dot(a, sel, preferred_element_type=jnp.float32)
    inv = 1.0 / jnp.maximum(cnt, 1.0)
    r2 = lax.broadcasted_iota(jnp.int32, (4, 64), 0)
    jc = lax.broadcasted_iota(jnp.int32, (4, 64), 1)
    exp = (jc // 16 == r2).astype(jnp.float32)
    nagg = a * jnp.dot(inv, exp, preferred_element_type=jnp.float32)
    h = (jnp.dot(x_ref[...], r_ref[...], preferred_element_type=jnp.float32)
         + jnp.dot(nagg, w_ref[...], preferred_element_type=jnp.float32))
    h_ref[...] = jnp.maximum(h, 0.0)
    inv_ref[...] = inv


def _tc2_body(h_ref, a_ref, inv_ref, bf_ref, r_ref, w_ref, b2_ref,
              out_ref, gs, gc):
    i = pl.program_id(0)
    a = a_ref[...]
    inv = inv_ref[...]
    r2 = lax.broadcasted_iota(jnp.int32, (4, 512), 0)
    jc = lax.broadcasted_iota(jnp.int32, (4, 512), 1)
    exp = (jc // 128 == r2).astype(jnp.float32)
    nagg = a * jnp.dot(inv, exp, preferred_element_type=jnp.float32)
    h2 = (jnp.dot(h_ref[...], r_ref[...], preferred_element_type=jnp.float32)
          + jnp.dot(nagg, w_ref[...], preferred_element_type=jnp.float32)
          + b2_ref[...])
    h2 = jnp.maximum(h2, 0.0)
    gids = lax.broadcasted_iota(jnp.int32, (1, _G), 1).astype(jnp.float32)
    oh = (bf_ref[...] == gids).astype(jnp.float32)
    gs_part = lax.dot_general(oh, h2, (((0,), (0,)), ((), ())),
                              preferred_element_type=jnp.float32)
    ones = jnp.ones((_BLK, 128), jnp.float32)
    gc_part = lax.dot_general(oh, ones, (((0,), (0,)), ((), ())),
                              preferred_element_type=jnp.float32)

    @pl.when(i == 0)
    def _():
        gs[...] = gs_part
        gc[...] = gc_part

    @pl.when(i > 0)
    def _():
        gs[...] += gs_part
        gc[...] += gc_part

    @pl.when(i == _NB - 1)
    def _():
        out_ref[...] = gs[...] / jnp.maximum(gc[...], 1.0)


def _tc_layer1(xp, agg1, root1p, w1cat):
    return pl.pallas_call(
        _tc1_body,
        grid=(_NB,),
        in_specs=[
            pl.BlockSpec((_BLK, 16), lambda i: (i, 0)),
            pl.BlockSpec((_BLK, 64), lambda i: (i, 0)),
            pl.BlockSpec((16, 128), lambda i: (0, 0)),
            pl.BlockSpec((64, 128), lambda i: (0, 0)),
        ],
        out_specs=[
            pl.BlockSpec((_BLK, 128), lambda i: (i, 0)),
            pl.BlockSpec((_BLK, 4), lambda i: (i, 0)),
        ],
        out_shape=[
            jax.ShapeDtypeStruct((_N, 128), jnp.float32),
            jax.ShapeDtypeStruct((_N, 4), jnp.float32),
        ],
    )(xp, agg1, root1p, w1cat)


def _tc_layer2(h, agg2, inv, batchf, root2, w2cat, b2):
    return pl.pallas_call(
        _tc2_body,
        grid=(_NB,),
        in_specs=[
            pl.BlockSpec((_BLK, 128), lambda i: (i, 0)),
            pl.BlockSpec((_BLK, 512), lambda i: (i, 0)),
            pl.BlockSpec((_BLK, 4), lambda i: (i, 0)),
            pl.BlockSpec((_BLK, 1), lambda i: (i, 0)),
            pl.BlockSpec((128, 128), lambda i: (0, 0)),
            pl.BlockSpec((512, 128), lambda i: (0, 0)),
            pl.BlockSpec((1, 128), lambda i: (0, 0)),
        ],
        out_specs=pl.BlockSpec((_G, 128), lambda i: (0, 0)),
        out_shape=jax.ShapeDtypeStruct((_G, 128), jnp.float32),
        scratch_shapes=[
            pltpu.VMEM((_G, 128), jnp.float32),
            pltpu.VMEM((_G, 128), jnp.float32),
        ],
    )(h, agg2, inv, batchf, root2, w2cat, b2)


def kernel(x, edge_index, edge_type, batch, W1, root1, b1, W2, root2, b2):
    src = edge_index[0]
    dst = edge_index[1]
    et = edge_type
    xp = jnp.concatenate([x, jnp.ones((_N, 1), x.dtype)], axis=1)
    root1p = jnp.concatenate([root1, b1[None, :]], axis=0)
    w1cat = jnp.pad(W1, ((0, 0), (0, 1), (0, 0))).reshape(64, 128)
    w2cat = W2.reshape(512, 128)
    batchf = batch.astype(jnp.float32).reshape(_N, 1)

    sc_agg1 = _make_sc_agg(16, 25088, 1)
    sc_agg2 = _make_sc_agg(128, 2560, 10)
    agg1 = sc_agg1(src, dst, et, xp)[:_N * 4].reshape(_N, 64)
    h, inv = _tc_layer1(xp, agg1, root1p, w1cat)
    agg2 = sc_agg2(src, dst, et, h)[:_N * 4].reshape(_N, 512)
    return _tc_layer2(h, agg2, inv, batchf, root2, w2cat,
                      b2.reshape(1, 128))
